# baseline (speedup 1.0000x reference)
import functools

import jax
import jax.numpy as jnp
from jax.experimental import pallas as pl
from jax.experimental.pallas import tpu as pltpu

VOCAB = 8192
DIM = 256
BM = 128


def _vote_kernel(h_ref, cb_ref, out_ref):
    h = h_ref[...]
    s = jnp.where(h < 0, -1.0, 1.0).astype(jnp.bfloat16)
    cb = cb_ref[...].astype(jnp.bfloat16)
    out_ref[...] = jax.lax.dot_general(
        s, cb, (((1,), (1,)), ((), ())), preferred_element_type=jnp.float32)


@jax.jit
def kernel(h, codebook):
    b, t, d = h.shape
    m = b * t
    h2 = h.reshape(m, d)
    grid = (m // BM,)
    out = pl.pallas_call(
        _vote_kernel,
        grid=grid,
        in_specs=[
            pl.BlockSpec((BM, d), lambda i: (i, 0)),
            pl.BlockSpec((VOCAB, d), lambda i: (0, 0)),
        ],
        out_specs=pl.BlockSpec((BM, VOCAB), lambda i: (i, 0)),
        out_shape=jax.ShapeDtypeStruct((m, VOCAB), jnp.float32),
        compiler_params=pltpu.CompilerParams(
            dimension_semantics=("parallel",),
        ),
    )(h2, codebook)
    return out.reshape(b, t, VOCAB)


# speedup vs baseline: 1.3350x; 1.3350x over previous
import jax
import jax.numpy as jnp
from jax.experimental import pallas as pl
from jax.experimental.pallas import tpu as pltpu

VOCAB = 8192
DIM = 256
BM = 256
NBUF = 4


def _vote_kernel(h_ref, cb_ref, out_ref, buf, sem):
    i = pl.program_id(0)
    nsteps = pl.num_programs(0)
    slot = jax.lax.rem(i, NBUF)

    @pl.when(i >= NBUF)
    def _wait_slot_free():
        pltpu.make_async_copy(
            buf.at[slot],
            out_ref.at[pl.ds((i - NBUF) * BM, BM), :],
            sem.at[slot],
        ).wait()

    s = jnp.where(h_ref[...] < 0, -1.0, 1.0).astype(jnp.bfloat16)
    cb = cb_ref[...].astype(jnp.bfloat16)
    buf[slot] = jax.lax.dot_general(
        s, cb, (((1,), (1,)), ((), ())), preferred_element_type=jnp.float32)
    pltpu.make_async_copy(
        buf.at[slot],
        out_ref.at[pl.ds(i * BM, BM), :],
        sem.at[slot],
    ).start()

    @pl.when(i == nsteps - 1)
    def _drain():
        for k in range(NBUF):
            step = i - (NBUF - 1) + k
            pltpu.make_async_copy(
                buf.at[jax.lax.rem(step, NBUF)],
                out_ref.at[pl.ds(step * BM, BM), :],
                sem.at[jax.lax.rem(step, NBUF)],
            ).wait()


@jax.jit
def kernel(h, codebook):
    b, t, d = h.shape
    m = b * t
    h2 = h.reshape(m, d)
    grid = (m // BM,)
    out = pl.pallas_call(
        _vote_kernel,
        grid=grid,
        in_specs=[
            pl.BlockSpec((BM, d), lambda i: (i, 0)),
            pl.BlockSpec((VOCAB, d), lambda i: (0, 0)),
        ],
        out_specs=pl.BlockSpec(memory_space=pl.ANY),
        out_shape=jax.ShapeDtypeStruct((m, VOCAB), jnp.float32),
        scratch_shapes=[
            pltpu.VMEM((NBUF, BM, VOCAB), jnp.float32),
            pltpu.SemaphoreType.DMA((NBUF,)),
        ],
        compiler_params=pltpu.CompilerParams(
            dimension_semantics=("arbitrary",),
        ),
    )(h2, codebook)
    return out.reshape(b, t, VOCAB)
